# Initial kernel scaffold; baseline (speedup 1.0000x reference)
import jax
import jax.numpy as jnp
from jax.experimental import pallas as pl


def kernel(pred, target):
    raise NotImplementedError("write your pallas kernel here")


# speedup vs baseline: 12.5309x; 12.5309x over previous
import functools

import jax
import jax.numpy as jnp
from jax import lax
from jax.experimental import pallas as pl
from jax.experimental.pallas import tpu as pltpu
from jax.experimental.pallas import tpu_sc as plsc

N = 16 * 512 * 512
K = max(int(N * (1.0 - 0.7)), max(1, 10000))
NB = 1024
SHIFT1, SHIFT2 = 21, 11
NW = 32
PER_W = N // NW
CH = 16384
NCH = PER_W // CH
HB = NB * 16


def _loss_body(x_ref, t_ref, o_ref):
    x = x_ref[...]
    t = t_ref[...]
    o_ref[...] = jnp.maximum(x, 0.0) - x * t + jnp.log1p(jnp.exp(-jnp.abs(x)))


def _loss(x2, t2):
    rows, cols = x2.shape
    blk = 512
    return pl.pallas_call(
        _loss_body,
        grid=(rows // blk,),
        in_specs=[pl.BlockSpec((blk, cols), lambda i: (i, 0))] * 2,
        out_specs=pl.BlockSpec((blk, cols), lambda i: (i, 0)),
        out_shape=jax.ShapeDtypeStruct((rows, cols), jnp.float32),
    )(x2, t2)


@functools.cache
def _make_sc_pass(prefix_shift, digit_shift):
    mesh = plsc.VectorSubcoreMesh(
        core_axis_name="c", subcore_axis_name="s", num_cores=2, num_subcores=16)

    @functools.partial(
        pl.kernel,
        out_type=(
            jax.ShapeDtypeStruct((NW, HB), jnp.int32),
            jax.ShapeDtypeStruct((NW, HB), jnp.float32),
        ),
        mesh=mesh,
        scratch_types=[
            pltpu.VMEM((CH,), jnp.float32),
            pltpu.VMEM((CH,), jnp.float32),
            pltpu.VMEM((HB,), jnp.int32),
            pltpu.VMEM((HB,), jnp.float32),
            pltpu.VMEM((16,), jnp.int32),
            pltpu.SemaphoreType.DMA,
            pltpu.SemaphoreType.DMA,
        ],
        compiler_params=pltpu.CompilerParams(needs_layout_passes=False),
    )
    def sc_pass(loss_hbm, sel_hbm, cnt_out, sum_out,
                buf0, buf1, cnt_h, sum_h, selv, sem0, sem1):
        wid = lax.axis_index("s") * 2 + lax.axis_index("c")
        base = wid * PER_W
        pltpu.sync_copy(sel_hbm, selv)
        sel = selv[...]

        zi = jnp.zeros((16,), jnp.int32)
        zf = jnp.zeros((16,), jnp.float32)

        def zero_body(j, carry):
            cnt_h[pl.ds(j * 16, 16)] = zi
            sum_h[pl.ds(j * 16, 16)] = zf
            return carry

        lax.fori_loop(0, HB // 16, zero_body, 0)

        lane = lax.iota(jnp.int32, 16)
        ones = jnp.ones((16,), jnp.int32)
        bufs, sems = (buf0, buf1), (sem0, sem1)
        copies = [None, None]
        copies[0] = pltpu.async_copy(loss_hbm.at[pl.ds(base, CH)], buf0, sem0)
        for g in range(NCH):
            if g + 1 < NCH:
                copies[(g + 1) % 2] = pltpu.async_copy(
                    loss_hbm.at[pl.ds(base + (g + 1) * CH, CH)],
                    bufs[(g + 1) % 2], sems[(g + 1) % 2])
            copies[g % 2].wait()
            buf = bufs[g % 2]

            def step(j, carry):
                v = buf[pl.ds(j * 16, 16)]
                b = plsc.bitcast(v, jnp.int32)
                m = lax.shift_right_logical(b, prefix_shift) == sel
                digit = lax.shift_right_logical(b, digit_shift) & (NB - 1)
                idx = digit * 16 + lane
                plsc.addupdate_scatter(cnt_h, [idx], ones, mask=m)
                plsc.addupdate_scatter(sum_h, [idx], v, mask=m)
                return carry

            lax.fori_loop(0, CH // 16, step, 0)

        pltpu.sync_copy(cnt_h, cnt_out.at[wid])
        pltpu.sync_copy(sum_h, sum_out.at[wid])

    return sc_pass


def _sc_pass1(loss, sel):
    return _make_sc_pass(31, SHIFT1)(loss, sel)


def _sc_pass2(loss, sel):
    return _make_sc_pass(SHIFT1, SHIFT2)(loss, sel)


def _suffix_counts(cg):
    ii = lax.broadcasted_iota(jnp.int32, (NB, NB), 0)
    jj = lax.broadcasted_iota(jnp.int32, (NB, NB), 1)
    return jnp.sum(jnp.where(jj >= ii, cg[None, :], 0), axis=1)


def _sel1_body(c_ref, s_ref, d1_ref, carry_ref, cacc, sacc):
    i = pl.program_id(0)

    @pl.when(i == 0)
    def _():
        cacc[...] = jnp.zeros((NB, 16), jnp.int32)
        sacc[...] = jnp.zeros((NB, 16), jnp.float32)

    cacc[...] += c_ref[0]
    sacc[...] += s_ref[0]

    @pl.when(i == NW - 1)
    def _():
        cg = jnp.sum(cacc[...], axis=1)
        sg = jnp.sum(sacc[...], axis=1)
        suffix = _suffix_counts(cg)
        d1 = jnp.sum((suffix >= K).astype(jnp.int32)) - 1
        io = lax.iota(jnp.int32, NB)
        c1 = jnp.sum(jnp.where(io > d1, cg, 0))
        s1 = jnp.sum(jnp.where(io > d1, sg, 0.0))
        d1_ref[...] = jnp.full((8, 128), d1, jnp.int32)
        ri = lax.broadcasted_iota(jnp.int32, (8, 128), 0)
        carry_ref[...] = jnp.where(
            ri == 0, s1, jnp.where(ri == 1, c1.astype(jnp.float32), 0.0))


def _sel1(c3, s3):
    return pl.pallas_call(
        _sel1_body,
        grid=(NW,),
        in_specs=[pl.BlockSpec((1, NB, 16), lambda i: (i, 0, 0))] * 2,
        out_specs=[pl.BlockSpec((8, 128), lambda i: (0, 0))] * 2,
        out_shape=[
            jax.ShapeDtypeStruct((8, 128), jnp.int32),
            jax.ShapeDtypeStruct((8, 128), jnp.float32),
        ],
        scratch_shapes=[
            pltpu.VMEM((NB, 16), jnp.int32),
            pltpu.VMEM((NB, 16), jnp.float32),
        ],
    )(c3, s3)


def _sel2_body(c_ref, s_ref, carry_ref, out_ref, cacc, sacc):
    i = pl.program_id(0)

    @pl.when(i == 0)
    def _():
        cacc[...] = jnp.zeros((NB, 16), jnp.int32)
        sacc[...] = jnp.zeros((NB, 16), jnp.float32)

    cacc[...] += c_ref[0]
    sacc[...] += s_ref[0]

    @pl.when(i == NW - 1)
    def _():
        cg = jnp.sum(cacc[...], axis=1)
        sg = jnp.sum(sacc[...], axis=1)
        s1 = carry_ref[0, 0]
        c1 = carry_ref[1, 0].astype(jnp.int32)
        k2 = K - c1
        suffix = _suffix_counts(cg)
        d2 = jnp.sum((suffix >= k2).astype(jnp.int32)) - 1
        io = lax.iota(jnp.int32, NB)
        c2 = jnp.sum(jnp.where(io > d2, cg, 0))
        s2 = jnp.sum(jnp.where(io > d2, sg, 0.0))
        nb = jnp.sum(jnp.where(io == d2, cg, 0))
        sb = jnp.sum(jnp.where(io == d2, sg, 0.0))
        r = (k2 - c2).astype(jnp.float32)
        ans = (s1 + s2 + r * (sb / nb.astype(jnp.float32))) * (1.0 / K)
        out_ref[...] = jnp.full((8, 128), ans, jnp.float32)


def _sel2(c3, s3, carry):
    return pl.pallas_call(
        _sel2_body,
        grid=(NW,),
        in_specs=[
            pl.BlockSpec((1, NB, 16), lambda i: (i, 0, 0)),
            pl.BlockSpec((1, NB, 16), lambda i: (i, 0, 0)),
            pl.BlockSpec((8, 128), lambda i: (0, 0)),
        ],
        out_specs=pl.BlockSpec((8, 128), lambda i: (0, 0)),
        out_shape=jax.ShapeDtypeStruct((8, 128), jnp.float32),
        scratch_shapes=[
            pltpu.VMEM((NB, 16), jnp.int32),
            pltpu.VMEM((NB, 16), jnp.float32),
        ],
    )(c3, s3, carry)


def kernel(pred, target):
    x2 = pred.reshape(4096, 1024)
    t2 = target.reshape(4096, 1024)
    loss = _loss(x2, t2).reshape(N)
    zsel = jnp.zeros((16,), jnp.int32)
    cnt1, sum1 = _sc_pass1(loss, zsel)
    d1_8, carry = _sel1(cnt1.reshape(NW, NB, 16), sum1.reshape(NW, NB, 16))
    d1v = d1_8.reshape(-1)[:16]
    cnt2, sum2 = _sc_pass2(loss, d1v)
    out = _sel2(cnt2.reshape(NW, NB, 16), sum2.reshape(NW, NB, 16), carry)
    return out[0, 0]


# speedup vs baseline: 12.6255x; 1.0075x over previous
import functools

import jax
import jax.numpy as jnp
from jax import lax
from jax.experimental import pallas as pl
from jax.experimental.pallas import tpu as pltpu
from jax.experimental.pallas import tpu_sc as plsc

N = 16 * 512 * 512
K = max(int(N * (1.0 - 0.7)), max(1, 10000))
NB = 1024
SHIFT1, SHIFT2 = 21, 11
NW = 32
PER_W = N // NW
CH = 16384
NCH = PER_W // CH
HB = NB * 16


def _loss_body(x_ref, t_ref, o_ref):
    x = x_ref[...]
    t = t_ref[...]
    o_ref[...] = jnp.maximum(x, 0.0) - x * t + jnp.log1p(jnp.exp(-jnp.abs(x)))


def _loss(x2, t2):
    rows, cols = x2.shape
    blk = 512
    return pl.pallas_call(
        _loss_body,
        grid=(rows // blk,),
        in_specs=[pl.BlockSpec((blk, cols), lambda i: (i, 0))] * 2,
        out_specs=pl.BlockSpec((blk, cols), lambda i: (i, 0)),
        out_shape=jax.ShapeDtypeStruct((rows, cols), jnp.float32),
    )(x2, t2)


@functools.cache
def _make_sc_pass(prefix_shift, digit_shift):
    mesh = plsc.VectorSubcoreMesh(
        core_axis_name="c", subcore_axis_name="s", num_cores=2, num_subcores=16)

    @functools.partial(
        pl.kernel,
        out_type=(
            jax.ShapeDtypeStruct((NW, HB), jnp.int32),
            jax.ShapeDtypeStruct((NW, HB), jnp.float32),
        ),
        mesh=mesh,
        scratch_types=[
            pltpu.VMEM((CH,), jnp.float32),
            pltpu.VMEM((CH,), jnp.float32),
            pltpu.VMEM((HB,), jnp.int32),
            pltpu.VMEM((HB,), jnp.float32),
            pltpu.VMEM((16,), jnp.int32),
            pltpu.SemaphoreType.DMA,
            pltpu.SemaphoreType.DMA,
        ],
        compiler_params=pltpu.CompilerParams(needs_layout_passes=False),
    )
    def sc_pass(loss_hbm, sel_hbm, cnt_out, sum_out,
                buf0, buf1, cnt_h, sum_h, selv, sem0, sem1):
        wid = lax.axis_index("s") * 2 + lax.axis_index("c")
        base = wid * PER_W
        pltpu.sync_copy(sel_hbm, selv)
        sel = selv[...]

        zi = jnp.zeros((16,), jnp.int32)
        zf = jnp.zeros((16,), jnp.float32)

        def zero_body(j, carry):
            for u in range(8):
                cnt_h[pl.ds(j * 128 + u * 16, 16)] = zi
                sum_h[pl.ds(j * 128 + u * 16, 16)] = zf
            return carry

        lax.fori_loop(0, HB // 128, zero_body, 0)

        lane = lax.iota(jnp.int32, 16)
        ones = jnp.ones((16,), jnp.int32)
        bufs, sems = (buf0, buf1), (sem0, sem1)
        copies = [None, None]
        copies[0] = pltpu.async_copy(loss_hbm.at[pl.ds(base, CH)], buf0, sem0)
        for g in range(NCH):
            if g + 1 < NCH:
                copies[(g + 1) % 2] = pltpu.async_copy(
                    loss_hbm.at[pl.ds(base + (g + 1) * CH, CH)],
                    bufs[(g + 1) % 2], sems[(g + 1) % 2])
            copies[g % 2].wait()
            buf = bufs[g % 2]

            def step(j, carry):
                for u in range(4):
                    v = buf[pl.ds(j * 64 + u * 16, 16)]
                    b = plsc.bitcast(v, jnp.int32)
                    m = lax.shift_right_logical(b, prefix_shift) == sel
                    digit = lax.shift_right_logical(b, digit_shift) & (NB - 1)
                    idx = digit * 16 + lane
                    plsc.addupdate_scatter(cnt_h, [idx], ones, mask=m)
                    plsc.addupdate_scatter(sum_h, [idx], v, mask=m)
                return carry

            lax.fori_loop(0, CH // 64, step, 0)

        pltpu.sync_copy(cnt_h, cnt_out.at[wid])
        pltpu.sync_copy(sum_h, sum_out.at[wid])

    return sc_pass


def _sc_pass1(loss, sel):
    return _make_sc_pass(31, SHIFT1)(loss, sel)


def _sc_pass2(loss, sel):
    return _make_sc_pass(SHIFT1, SHIFT2)(loss, sel)


def _suffix_counts(cg):
    ii = lax.broadcasted_iota(jnp.int32, (NB, NB), 0)
    jj = lax.broadcasted_iota(jnp.int32, (NB, NB), 1)
    return jnp.sum(jnp.where(jj >= ii, cg[None, :], 0), axis=1)


def _sel1_body(c_ref, s_ref, d1_ref, carry_ref, cacc, sacc):
    i = pl.program_id(0)

    @pl.when(i == 0)
    def _():
        cacc[...] = jnp.zeros((NB, 16), jnp.int32)
        sacc[...] = jnp.zeros((NB, 16), jnp.float32)

    cacc[...] += c_ref[0]
    sacc[...] += s_ref[0]

    @pl.when(i == NW - 1)
    def _():
        cg = jnp.sum(cacc[...], axis=1)
        sg = jnp.sum(sacc[...], axis=1)
        suffix = _suffix_counts(cg)
        d1 = jnp.sum((suffix >= K).astype(jnp.int32)) - 1
        io = lax.iota(jnp.int32, NB)
        c1 = jnp.sum(jnp.where(io > d1, cg, 0))
        s1 = jnp.sum(jnp.where(io > d1, sg, 0.0))
        d1_ref[...] = jnp.full((8, 128), d1, jnp.int32)
        ri = lax.broadcasted_iota(jnp.int32, (8, 128), 0)
        carry_ref[...] = jnp.where(
            ri == 0, s1, jnp.where(ri == 1, c1.astype(jnp.float32), 0.0))


def _sel1(c3, s3):
    return pl.pallas_call(
        _sel1_body,
        grid=(NW,),
        in_specs=[pl.BlockSpec((1, NB, 16), lambda i: (i, 0, 0))] * 2,
        out_specs=[pl.BlockSpec((8, 128), lambda i: (0, 0))] * 2,
        out_shape=[
            jax.ShapeDtypeStruct((8, 128), jnp.int32),
            jax.ShapeDtypeStruct((8, 128), jnp.float32),
        ],
        scratch_shapes=[
            pltpu.VMEM((NB, 16), jnp.int32),
            pltpu.VMEM((NB, 16), jnp.float32),
        ],
    )(c3, s3)


def _sel2_body(c_ref, s_ref, carry_ref, out_ref, cacc, sacc):
    i = pl.program_id(0)

    @pl.when(i == 0)
    def _():
        cacc[...] = jnp.zeros((NB, 16), jnp.int32)
        sacc[...] = jnp.zeros((NB, 16), jnp.float32)

    cacc[...] += c_ref[0]
    sacc[...] += s_ref[0]

    @pl.when(i == NW - 1)
    def _():
        cg = jnp.sum(cacc[...], axis=1)
        sg = jnp.sum(sacc[...], axis=1)
        s1 = carry_ref[0, 0]
        c1 = carry_ref[1, 0].astype(jnp.int32)
        k2 = K - c1
        suffix = _suffix_counts(cg)
        d2 = jnp.sum((suffix >= k2).astype(jnp.int32)) - 1
        io = lax.iota(jnp.int32, NB)
        c2 = jnp.sum(jnp.where(io > d2, cg, 0))
        s2 = jnp.sum(jnp.where(io > d2, sg, 0.0))
        nb = jnp.sum(jnp.where(io == d2, cg, 0))
        sb = jnp.sum(jnp.where(io == d2, sg, 0.0))
        r = (k2 - c2).astype(jnp.float32)
        ans = (s1 + s2 + r * (sb / nb.astype(jnp.float32))) * (1.0 / K)
        out_ref[...] = jnp.full((8, 128), ans, jnp.float32)


def _sel2(c3, s3, carry):
    return pl.pallas_call(
        _sel2_body,
        grid=(NW,),
        in_specs=[
            pl.BlockSpec((1, NB, 16), lambda i: (i, 0, 0)),
            pl.BlockSpec((1, NB, 16), lambda i: (i, 0, 0)),
            pl.BlockSpec((8, 128), lambda i: (0, 0)),
        ],
        out_specs=pl.BlockSpec((8, 128), lambda i: (0, 0)),
        out_shape=jax.ShapeDtypeStruct((8, 128), jnp.float32),
        scratch_shapes=[
            pltpu.VMEM((NB, 16), jnp.int32),
            pltpu.VMEM((NB, 16), jnp.float32),
        ],
    )(c3, s3, carry)


def kernel(pred, target):
    x2 = pred.reshape(4096, 1024)
    t2 = target.reshape(4096, 1024)
    loss = _loss(x2, t2).reshape(N)
    zsel = jnp.zeros((16,), jnp.int32)
    cnt1, sum1 = _sc_pass1(loss, zsel)
    d1_8, carry = _sel1(cnt1.reshape(NW, NB, 16), sum1.reshape(NW, NB, 16))
    d1v = d1_8.reshape(-1)[:16]
    cnt2, sum2 = _sc_pass2(loss, d1v)
    out = _sel2(cnt2.reshape(NW, NB, 16), sum2.reshape(NW, NB, 16), carry)
    return out[0, 0]


# speedup vs baseline: 29.1022x; 2.3050x over previous
import functools

import jax
import jax.numpy as jnp
from jax import lax
from jax.experimental import pallas as pl
from jax.experimental.pallas import tpu as pltpu
from jax.experimental.pallas import tpu_sc as plsc

N = 16 * 512 * 512
K = max(int(N * (1.0 - 0.7)), max(1, 10000))
NB = 1024
SHIFT1, SHIFT2 = 21, 11
NW = 32
PER_W = N // NW
CH = 16384
NCH = PER_W // CH
HB = NB * 16


def _loss_body(x_ref, t_ref, o_ref):
    x = x_ref[...]
    t = t_ref[...]
    o_ref[...] = jnp.maximum(x, 0.0) - x * t + jnp.log1p(jnp.exp(-jnp.abs(x)))


def _loss(x2, t2):
    rows, cols = x2.shape
    blk = 512
    return pl.pallas_call(
        _loss_body,
        grid=(rows // blk,),
        in_specs=[pl.BlockSpec((blk, cols), lambda i: (i, 0))] * 2,
        out_specs=pl.BlockSpec((blk, cols), lambda i: (i, 0)),
        out_shape=jax.ShapeDtypeStruct((rows, cols), jnp.float32),
    )(x2, t2)


@functools.cache
def _make_sc_pass(prefix_shift, digit_shift):
    mesh = plsc.VectorSubcoreMesh(
        core_axis_name="c", subcore_axis_name="s", num_cores=2, num_subcores=16)

    @functools.partial(
        pl.kernel,
        out_type=(
            jax.ShapeDtypeStruct((NW, NB), jnp.int32),
            jax.ShapeDtypeStruct((NW, NB), jnp.float32),
        ),
        mesh=mesh,
        scratch_types=[
            pltpu.VMEM((CH,), jnp.float32),
            pltpu.VMEM((CH,), jnp.float32),
            pltpu.VMEM((HB,), jnp.int32),
            pltpu.VMEM((HB,), jnp.float32),
            pltpu.VMEM((NB,), jnp.int32),
            pltpu.VMEM((NB,), jnp.float32),
            pltpu.VMEM((16,), jnp.int32),
            pltpu.SemaphoreType.DMA,
            pltpu.SemaphoreType.DMA,
        ],
        compiler_params=pltpu.CompilerParams(needs_layout_passes=False),
    )
    def sc_pass(loss_hbm, sel_hbm, cnt_out, sum_out,
                buf0, buf1, cnt_h, sum_h, red_c, red_s, selv, sem0, sem1):
        wid = lax.axis_index("s") * 2 + lax.axis_index("c")
        base = wid * PER_W
        pltpu.sync_copy(sel_hbm, selv)
        sel = selv[...]

        zi = jnp.zeros((16,), jnp.int32)
        zf = jnp.zeros((16,), jnp.float32)

        def zero_body(j, carry):
            for u in range(8):
                cnt_h[pl.ds(j * 128 + u * 16, 16)] = zi
                sum_h[pl.ds(j * 128 + u * 16, 16)] = zf
            return carry

        lax.fori_loop(0, HB // 128, zero_body, 0)

        lane = lax.iota(jnp.int32, 16)
        ones = jnp.ones((16,), jnp.int32)
        bufs, sems = (buf0, buf1), (sem0, sem1)
        copies = [None, None]
        copies[0] = pltpu.async_copy(loss_hbm.at[pl.ds(base, CH)], buf0, sem0)
        for g in range(NCH):
            if g + 1 < NCH:
                copies[(g + 1) % 2] = pltpu.async_copy(
                    loss_hbm.at[pl.ds(base + (g + 1) * CH, CH)],
                    bufs[(g + 1) % 2], sems[(g + 1) % 2])
            copies[g % 2].wait()
            buf = bufs[g % 2]

            def step(j, carry):
                for u in range(4):
                    v = buf[pl.ds(j * 64 + u * 16, 16)]
                    b = plsc.bitcast(v, jnp.int32)
                    m = lax.shift_right_logical(b, prefix_shift) == sel
                    digit = lax.shift_right_logical(b, digit_shift) & (NB - 1)
                    idx = digit * 16 + lane
                    plsc.addupdate_scatter(cnt_h, [idx], ones, mask=m)
                    plsc.addupdate_scatter(sum_h, [idx], v, mask=m)
                return carry

            lax.fori_loop(0, CH // 64, step, 0)

        def red_body(cidx, carry):
            dig0 = (cidx * 16 + lane) * 16
            accc = plsc.load_gather(cnt_h, [dig0])
            accs = plsc.load_gather(sum_h, [dig0])
            for l in range(1, 16):
                accc += plsc.load_gather(cnt_h, [dig0 + l])
                accs += plsc.load_gather(sum_h, [dig0 + l])
            red_c[pl.ds(cidx * 16, 16)] = accc
            red_s[pl.ds(cidx * 16, 16)] = accs
            return carry

        lax.fori_loop(0, NB // 16, red_body, 0)
        pltpu.sync_copy(red_c, cnt_out.at[wid])
        pltpu.sync_copy(red_s, sum_out.at[wid])

    return sc_pass


def _sc_pass1(loss, sel):
    return _make_sc_pass(31, SHIFT1)(loss, sel)


def _sc_pass2(loss, sel):
    return _make_sc_pass(SHIFT1, SHIFT2)(loss, sel)


def _suffix_counts(cg):
    ii = lax.broadcasted_iota(jnp.int32, (NB, NB), 0)
    jj = lax.broadcasted_iota(jnp.int32, (NB, NB), 1)
    return jnp.sum(jnp.where(jj >= ii, cg[None, :], 0), axis=1)


def _sel1_body(c_ref, s_ref, d1_ref, carry_ref):
    cg = jnp.sum(c_ref[...], axis=0)
    sg = jnp.sum(s_ref[...], axis=0)
    suffix = _suffix_counts(cg)
    d1 = jnp.sum((suffix >= K).astype(jnp.int32)) - 1
    io = lax.iota(jnp.int32, NB)
    c1 = jnp.sum(jnp.where(io > d1, cg, 0))
    s1 = jnp.sum(jnp.where(io > d1, sg, 0.0))
    d1_ref[...] = jnp.full((8, 128), d1, jnp.int32)
    ri = lax.broadcasted_iota(jnp.int32, (8, 128), 0)
    carry_ref[...] = jnp.where(
        ri == 0, s1, jnp.where(ri == 1, c1.astype(jnp.float32), 0.0))


def _sel1(c2d, s2d):
    return pl.pallas_call(
        _sel1_body,
        out_shape=[
            jax.ShapeDtypeStruct((8, 128), jnp.int32),
            jax.ShapeDtypeStruct((8, 128), jnp.float32),
        ],
    )(c2d, s2d)


def _sel2_body(c_ref, s_ref, carry_ref, out_ref):
    cg = jnp.sum(c_ref[...], axis=0)
    sg = jnp.sum(s_ref[...], axis=0)
    s1 = carry_ref[0, 0]
    c1 = carry_ref[1, 0].astype(jnp.int32)
    k2 = K - c1
    suffix = _suffix_counts(cg)
    d2 = jnp.sum((suffix >= k2).astype(jnp.int32)) - 1
    io = lax.iota(jnp.int32, NB)
    c2 = jnp.sum(jnp.where(io > d2, cg, 0))
    s2 = jnp.sum(jnp.where(io > d2, sg, 0.0))
    nb = jnp.sum(jnp.where(io == d2, cg, 0))
    sb = jnp.sum(jnp.where(io == d2, sg, 0.0))
    r = (k2 - c2).astype(jnp.float32)
    ans = (s1 + s2 + r * (sb / nb.astype(jnp.float32))) * (1.0 / K)
    out_ref[...] = jnp.full((8, 128), ans, jnp.float32)


def _sel2(c2d, s2d, carry):
    return pl.pallas_call(
        _sel2_body,
        out_shape=jax.ShapeDtypeStruct((8, 128), jnp.float32),
    )(c2d, s2d, carry)


def kernel(pred, target):
    x2 = pred.reshape(4096, 1024)
    t2 = target.reshape(4096, 1024)
    loss = _loss(x2, t2).reshape(N)
    zsel = jnp.zeros((16,), jnp.int32)
    cnt1, sum1 = _sc_pass1(loss, zsel)
    d1_8, carry = _sel1(cnt1, sum1)
    d1v = d1_8.reshape(-1)[:16]
    cnt2, sum2 = _sc_pass2(loss, d1v)
    out = _sel2(cnt2, sum2, carry)
    return out[0, 0]
